# Initial kernel scaffold; baseline (speedup 1.0000x reference)
import functools

import jax
import jax.numpy as jnp
from jax import lax
from jax.experimental import pallas as pl
from jax.experimental.pallas import tpu as pltpu
from jax.experimental.pallas import tpu_sc as plsc

N = 10000
E = 320000
IN_DIM = 128
HID = 64
OUT = 16
EPS = 1e-5

NP = 10240
FILL = N
D1 = 80
D2 = OUT
B = 128
NC = 2
NS = 16
NWORK = NC * NS
G = 79
E_PAD = NWORK * G * B
ROWS_PER_TILE = NP // NS
ZB = ROWS_PER_TILE // B


def _make_sc_aggregate(d):
    mesh = plsc.VectorSubcoreMesh(core_axis_name="c", subcore_axis_name="s")

    def body(p_hbm, src_hbm, dst_hbm, z_hbm, out_hbm,
             src_v, dst_v, rows_v, zbuf_v, acc_sh, sem):
        cid = lax.axis_index("c")
        sid = lax.axis_index("s")
        tile = cid * NS + sid

        pltpu.sync_copy(z_hbm, zbuf_v)
        for j in range(ZB):
            pltpu.sync_copy(zbuf_v, acc_sh.at[pl.ds(sid * ROWS_PER_TILE + j * B, B)])
        plsc.subcore_barrier()

        base = tile * G
        pltpu.sync_copy(src_hbm.at[pl.ds(base, G)], src_v)
        pltpu.sync_copy(dst_hbm.at[pl.ds(base, G)], dst_v)

        def step(g, carry):
            pltpu.async_copy(p_hbm.at[src_v.at[g]], rows_v, sem).wait()
            pltpu.sync_copy(rows_v, acc_sh.at[dst_v.at[g]], add=True)
            return carry

        lax.fori_loop(0, G, step, 0)
        plsc.subcore_barrier()

        for j in range(ZB):
            r0 = sid * ROWS_PER_TILE + j * B
            pltpu.sync_copy(acc_sh.at[pl.ds(r0, B)], rows_v)
            pltpu.sync_copy(rows_v, out_hbm.at[cid, pl.ds(r0, B)])

    return functools.partial(
        pl.kernel,
        out_type=jax.ShapeDtypeStruct((NC, NP, d), jnp.float32),
        mesh=mesh,
        scratch_types=[
            pltpu.VMEM((G, B), jnp.int32),
            pltpu.VMEM((G, B), jnp.int32),
            pltpu.VMEM((B, d), jnp.float32),
            pltpu.VMEM((B, d), jnp.float32),
            pltpu.VMEM_SHARED((NP, d), jnp.float32),
            pltpu.SemaphoreType.DMA,
        ],
    )(body)


_sc_agg1 = _make_sc_aggregate(D1)
_sc_agg2 = _make_sc_aggregate(D2)

_DN = (((1,), (1,)), ((), ()))


def _tc_pre(x_ref, wl_ref, wr_ref, p1_ref, r1_ref):
    x = x_ref[...]
    xw = lax.dot_general(x, wl_ref[...], _DN, preferred_element_type=jnp.float32)
    r1_ref[...] = lax.dot_general(x, wr_ref[...], _DN,
                                  preferred_element_type=jnp.float32)
    ones = jnp.ones((NP, 1), jnp.float32)
    zeros = jnp.zeros((NP, D1 - HID - 1), jnp.float32)
    p1_ref[...] = jnp.concatenate([xw, ones, zeros], axis=1)


def _tc_mid(part_ref, r1_ref, bl1_ref, gamma_ref, beta_ref, wl2_ref, wr2_ref,
            p2_ref, r2_ref, invdeg_ref):
    acc = part_ref[0] + part_ref[1]
    deg = acc[:, HID:HID + 1]
    invdeg = 1.0 / jnp.maximum(deg, 1.0)
    invdeg_ref[...] = invdeg
    h = acc[:, :HID] * invdeg + bl1_ref[...] + r1_ref[...]
    rows = lax.broadcasted_iota(jnp.int32, (NP, 1), 0)
    mask = rows < N
    hm = jnp.where(mask, h, 0.0)
    mean = jnp.sum(hm, axis=0, keepdims=True) * (1.0 / N)
    cm = jnp.where(mask, h - mean, 0.0)
    var = jnp.sum(cm * cm, axis=0, keepdims=True) * (1.0 / N)
    hn = (h - mean) * lax.rsqrt(var + EPS) * gamma_ref[...] + beta_ref[...]
    hr = jnp.maximum(hn, 0.0)
    p2_ref[...] = lax.dot_general(hr, wl2_ref[...], _DN,
                                  preferred_element_type=jnp.float32)
    r2_ref[...] = lax.dot_general(hr, wr2_ref[...], _DN,
                                  preferred_element_type=jnp.float32)


def _tc_post(part2_ref, r2_ref, invdeg_ref, bl2_ref, out_ref):
    acc = part2_ref[0] + part2_ref[1]
    out_ref[...] = acc * invdeg_ref[...] + bl2_ref[...] + r2_ref[...]


def kernel(x, edge_index, Wl1, bl1, Wr1, gamma, beta, Wl2, bl2, Wr2):
    xp = jnp.pad(x, ((0, NP - N), (0, 0)))
    src = edge_index[0].astype(jnp.int32)
    dst = edge_index[1].astype(jnp.int32)
    fill = jnp.full((E_PAD - E,), FILL, jnp.int32)
    src2d = jnp.concatenate([src, fill]).reshape(NWORK * G, B)
    dst2d = jnp.concatenate([dst, fill]).reshape(NWORK * G, B)
    z1 = jnp.zeros((B, D1), jnp.float32)
    z2 = jnp.zeros((B, D2), jnp.float32)
    bl1r = bl1.reshape(1, HID)
    gammar = gamma.reshape(1, HID)
    betar = beta.reshape(1, HID)
    bl2r = bl2.reshape(1, OUT)

    p1ext, r1 = pl.pallas_call(
        _tc_pre,
        out_shape=[jax.ShapeDtypeStruct((NP, D1), jnp.float32),
                   jax.ShapeDtypeStruct((NP, HID), jnp.float32)],
    )(xp, Wl1, Wr1)

    part1 = _sc_agg1(p1ext, src2d, dst2d, z1)

    p2, r2, invdeg = pl.pallas_call(
        _tc_mid,
        out_shape=[jax.ShapeDtypeStruct((NP, D2), jnp.float32),
                   jax.ShapeDtypeStruct((NP, D2), jnp.float32),
                   jax.ShapeDtypeStruct((NP, 1), jnp.float32)],
    )(part1, r1, bl1r, gammar, betar, Wl2, Wr2)

    part2 = _sc_agg2(p2, src2d, dst2d, z2)

    out = pl.pallas_call(
        _tc_post,
        out_shape=jax.ShapeDtypeStruct((NP, D2), jnp.float32),
    )(part2, r2, invdeg, bl2r)
    return out[:N]


# speedup vs baseline: 5.3107x; 5.3107x over previous
import functools

import jax
import jax.numpy as jnp
from jax import lax
from jax.experimental import pallas as pl
from jax.experimental.pallas import tpu as pltpu
from jax.experimental.pallas import tpu_sc as plsc

N = 10000
E = 320000
IN_DIM = 128
HID = 64
OUT = 16
EPS = 1e-5

NP = 10240
FILL = N
D1 = 80
D2 = OUT
B = 128
NC = 2
NS = 16
NWORK = NC * NS
G = 80
E_PAD = NWORK * G * B
ROWS_PER_TILE = NP // NS
ZB = ROWS_PER_TILE // B


def _make_sc_aggregate(d):
    mesh = plsc.VectorSubcoreMesh(core_axis_name="c", subcore_axis_name="s")

    def body(p_hbm, src_hbm, dst_hbm, z_hbm, out_hbm,
             src_v, dst_v, rows_v, zbuf_v, acc_sh, sem):
        cid = lax.axis_index("c")
        sid = lax.axis_index("s")
        tile = cid * NS + sid

        pltpu.sync_copy(z_hbm, zbuf_v)
        for j in range(ZB):
            pltpu.sync_copy(zbuf_v, acc_sh.at[pl.ds(sid * ROWS_PER_TILE + j * B, B)])
        plsc.subcore_barrier()

        base = tile * G
        pltpu.sync_copy(src_hbm.at[pl.ds(base, G)], src_v)
        pltpu.sync_copy(dst_hbm.at[pl.ds(base, G)], dst_v)

        def step(g, carry):
            pltpu.async_copy(p_hbm.at[src_v.at[g]], rows_v, sem).wait()
            pltpu.sync_copy(rows_v, acc_sh.at[dst_v.at[g]], add=True)
            return carry

        lax.fori_loop(0, G, step, 0)
        plsc.subcore_barrier()

        for j in range(ZB):
            r0 = sid * ROWS_PER_TILE + j * B
            pltpu.sync_copy(acc_sh.at[pl.ds(r0, B)], rows_v)
            pltpu.sync_copy(rows_v, out_hbm.at[cid, pl.ds(r0, B)])

    return functools.partial(
        pl.kernel,
        out_type=jax.ShapeDtypeStruct((NC, NP, d), jnp.float32),
        mesh=mesh,
        scratch_types=[
            pltpu.VMEM((G, B), jnp.int32),
            pltpu.VMEM((G, B), jnp.int32),
            pltpu.VMEM((B, d), jnp.float32),
            pltpu.VMEM((B, d), jnp.float32),
            pltpu.VMEM_SHARED((NP, d), jnp.float32),
            pltpu.SemaphoreType.DMA,
        ],
        compiler_params=pltpu.CompilerParams(use_tc_tiling_on_sc=False),
    )(body)


_sc_agg1 = _make_sc_aggregate(D1)
_sc_agg2 = _make_sc_aggregate(D2)

_DN = (((1,), (1,)), ((), ()))


def _tc_pre(x_ref, wl_ref, wr_ref, p1_ref, r1_ref):
    x = x_ref[...]
    xw = lax.dot_general(x, wl_ref[...], _DN, preferred_element_type=jnp.float32)
    r1_ref[...] = lax.dot_general(x, wr_ref[...], _DN,
                                  preferred_element_type=jnp.float32)
    ones = jnp.ones((NP, 1), jnp.float32)
    zeros = jnp.zeros((NP, D1 - HID - 1), jnp.float32)
    p1_ref[...] = jnp.concatenate([xw, ones, zeros], axis=1)


def _tc_mid(part_ref, r1_ref, bl1_ref, gamma_ref, beta_ref, wl2_ref, wr2_ref,
            p2_ref, r2_ref, invdeg_ref):
    acc = part_ref[0] + part_ref[1]
    deg = acc[:, HID:HID + 1]
    invdeg = 1.0 / jnp.maximum(deg, 1.0)
    invdeg_ref[...] = invdeg
    h = acc[:, :HID] * invdeg + bl1_ref[...] + r1_ref[...]
    rows = lax.broadcasted_iota(jnp.int32, (NP, 1), 0)
    mask = rows < N
    hm = jnp.where(mask, h, 0.0)
    mean = jnp.sum(hm, axis=0, keepdims=True) * (1.0 / N)
    cm = jnp.where(mask, h - mean, 0.0)
    var = jnp.sum(cm * cm, axis=0, keepdims=True) * (1.0 / N)
    hn = (h - mean) * lax.rsqrt(var + EPS) * gamma_ref[...] + beta_ref[...]
    hr = jnp.maximum(hn, 0.0)
    p2_ref[...] = lax.dot_general(hr, wl2_ref[...], _DN,
                                  preferred_element_type=jnp.float32)
    r2_ref[...] = lax.dot_general(hr, wr2_ref[...], _DN,
                                  preferred_element_type=jnp.float32)


def _tc_post(part2_ref, r2_ref, invdeg_ref, bl2_ref, out_ref):
    acc = part2_ref[0] + part2_ref[1]
    out_ref[...] = acc * invdeg_ref[...] + bl2_ref[...] + r2_ref[...]


def kernel(x, edge_index, Wl1, bl1, Wr1, gamma, beta, Wl2, bl2, Wr2):
    xp = jnp.pad(x, ((0, NP - N), (0, 0)))
    src = edge_index[0].astype(jnp.int32)
    dst = edge_index[1].astype(jnp.int32)
    fill = jnp.full((E_PAD - E,), FILL, jnp.int32)
    src2d = jnp.concatenate([src, fill]).reshape(NWORK * G, B)
    dst2d = jnp.concatenate([dst, fill]).reshape(NWORK * G, B)
    z1 = jnp.zeros((B, D1), jnp.float32)
    z2 = jnp.zeros((B, D2), jnp.float32)
    bl1r = bl1.reshape(1, HID)
    gammar = gamma.reshape(1, HID)
    betar = beta.reshape(1, HID)
    bl2r = bl2.reshape(1, OUT)

    p1ext, r1 = pl.pallas_call(
        _tc_pre,
        out_shape=[jax.ShapeDtypeStruct((NP, D1), jnp.float32),
                   jax.ShapeDtypeStruct((NP, HID), jnp.float32)],
    )(xp, Wl1, Wr1)

    part1 = _sc_agg1(p1ext, src2d, dst2d, z1)

    p2, r2, invdeg = pl.pallas_call(
        _tc_mid,
        out_shape=[jax.ShapeDtypeStruct((NP, D2), jnp.float32),
                   jax.ShapeDtypeStruct((NP, D2), jnp.float32),
                   jax.ShapeDtypeStruct((NP, 1), jnp.float32)],
    )(part1, r1, bl1r, gammar, betar, Wl2, Wr2)

    part2 = _sc_agg2(p2, src2d, dst2d, z2)

    out = pl.pallas_call(
        _tc_post,
        out_shape=jax.ShapeDtypeStruct((NP, D2), jnp.float32),
    )(part2, r2, invdeg, bl2r)
    return out[:N]


# speedup vs baseline: 11.5999x; 2.1843x over previous
import functools

import jax
import jax.numpy as jnp
from jax import lax
from jax.experimental import pallas as pl
from jax.experimental.pallas import tpu as pltpu
from jax.experimental.pallas import tpu_sc as plsc

N = 10000
E = 320000
IN_DIM = 128
HID = 64
OUT = 16
EPS = 1e-5

NP = 10240
FILL = N
D1 = 80
D2 = OUT
B = 128
NC = 2
NS = 16
NWORK = NC * NS
G = 80
E_PAD = NWORK * G * B
ROWS_PER_TILE = NP // NS
ZB = ROWS_PER_TILE // B


def _make_sc_aggregate(d):
    mesh = plsc.VectorSubcoreMesh(core_axis_name="c", subcore_axis_name="s")

    def body(p_hbm, src_hbm, dst_hbm, z_hbm, out_hbm,
             src_v, dst_v, rows_v, zbuf_v, acc_sh, sem):
        cid = lax.axis_index("c")
        sid = lax.axis_index("s")
        tile = cid * NS + sid

        pltpu.sync_copy(z_hbm, zbuf_v)
        for j in range(ZB):
            pltpu.sync_copy(zbuf_v, acc_sh.at[pl.ds(sid * ROWS_PER_TILE + j * B, B)])
        plsc.subcore_barrier()

        base = tile * G
        pltpu.sync_copy(src_hbm.at[pl.ds(base, G)], src_v)
        pltpu.sync_copy(dst_hbm.at[pl.ds(base, G)], dst_v)

        def step(g, carry):
            pltpu.async_copy(p_hbm.at[src_v.at[g]], rows_v, sem).wait()
            pltpu.sync_copy(rows_v, acc_sh.at[dst_v.at[g]], add=True)
            return carry

        lax.fori_loop(0, G, step, 0)
        plsc.subcore_barrier()

        for j in range(ZB):
            r0 = sid * ROWS_PER_TILE + j * B
            pltpu.sync_copy(acc_sh.at[pl.ds(r0, B)], rows_v)
            pltpu.sync_copy(rows_v, out_hbm.at[cid, pl.ds(r0, B)])

    return functools.partial(
        pl.kernel,
        out_type=jax.ShapeDtypeStruct((NC, NP, d), jnp.float32),
        mesh=mesh,
        scratch_types=[
            pltpu.VMEM((G, B), jnp.int32),
            pltpu.VMEM((G, B), jnp.int32),
            pltpu.VMEM((B, d), jnp.float32),
            pltpu.VMEM((B, d), jnp.float32),
            pltpu.VMEM_SHARED((NP, d), jnp.float32),
            pltpu.SemaphoreType.DMA,
        ],
        compiler_params=pltpu.CompilerParams(use_tc_tiling_on_sc=False),
    )(body)


_sc_agg1 = _make_sc_aggregate(D1)
_sc_agg2 = _make_sc_aggregate(D2)

_DN = (((1,), (1,)), ((), ()))


def _tc_pre(x_ref, wl_ref, wr_ref, p1_ref, r1_ref):
    x = x_ref[...]
    xw = lax.dot_general(x, wl_ref[...], _DN, preferred_element_type=jnp.float32)
    r1_ref[...] = lax.dot_general(x, wr_ref[...], _DN,
                                  preferred_element_type=jnp.float32)
    ones = jnp.ones((NP, 1), jnp.float32)
    zeros = jnp.zeros((NP, D1 - HID - 1), jnp.float32)
    p1_ref[...] = jnp.concatenate([xw, ones, zeros], axis=1)


def _tc_mid(part_ref, r1_ref, bl1_ref, gamma_ref, beta_ref, wl2_ref, wr2_ref,
            p2_ref, r2_ref, invdeg_ref):
    acc = part_ref[0] + part_ref[1]
    deg = acc[:, HID:HID + 1]
    invdeg = 1.0 / jnp.maximum(deg, 1.0)
    invdeg_ref[...] = invdeg
    h = acc[:, :HID] * invdeg + bl1_ref[...] + r1_ref[...]
    rows = lax.broadcasted_iota(jnp.int32, (NP, 1), 0)
    mask = rows < N
    hm = jnp.where(mask, h, 0.0)
    mean = jnp.sum(hm, axis=0, keepdims=True) * (1.0 / N)
    cm = jnp.where(mask, h - mean, 0.0)
    var = jnp.sum(cm * cm, axis=0, keepdims=True) * (1.0 / N)
    hn = (h - mean) * lax.rsqrt(var + EPS) * gamma_ref[...] + beta_ref[...]
    hr = jnp.maximum(hn, 0.0)
    p2_ref[...] = lax.dot_general(hr, wl2_ref[...], _DN,
                                  preferred_element_type=jnp.float32)
    r2_ref[...] = lax.dot_general(hr, wr2_ref[...], _DN,
                                  preferred_element_type=jnp.float32)


def _tc_post(part2_ref, r2_ref, invdeg_ref, bl2_ref, out_ref):
    acc = part2_ref[0] + part2_ref[1]
    out_ref[...] = acc * invdeg_ref[...] + bl2_ref[...] + r2_ref[...]


def kernel(x, edge_index, Wl1, bl1, Wr1, gamma, beta, Wl2, bl2, Wr2):
    xp = jnp.pad(x, ((0, NP - N), (0, 0)))
    src = edge_index[0].astype(jnp.int32)
    dst = edge_index[1].astype(jnp.int32)
    fill = N + (jnp.arange(E_PAD - E, dtype=jnp.int32) % (NP - N))
    src2d = jnp.concatenate([src, fill]).reshape(NWORK * G, B)
    dst2d = jnp.concatenate([dst, fill]).reshape(NWORK * G, B)
    z1 = jnp.zeros((B, D1), jnp.float32)
    z2 = jnp.zeros((B, D2), jnp.float32)
    bl1r = bl1.reshape(1, HID)
    gammar = gamma.reshape(1, HID)
    betar = beta.reshape(1, HID)
    bl2r = bl2.reshape(1, OUT)

    p1ext, r1 = pl.pallas_call(
        _tc_pre,
        out_shape=[jax.ShapeDtypeStruct((NP, D1), jnp.float32),
                   jax.ShapeDtypeStruct((NP, HID), jnp.float32)],
    )(xp, Wl1, Wr1)

    part1 = _sc_agg1(p1ext, src2d, dst2d, z1)

    p2, r2, invdeg = pl.pallas_call(
        _tc_mid,
        out_shape=[jax.ShapeDtypeStruct((NP, D2), jnp.float32),
                   jax.ShapeDtypeStruct((NP, D2), jnp.float32),
                   jax.ShapeDtypeStruct((NP, 1), jnp.float32)],
    )(part1, r1, bl1r, gammar, betar, Wl2, Wr2)

    part2 = _sc_agg2(p2, src2d, dst2d, z2)

    out = pl.pallas_call(
        _tc_post,
        out_shape=jax.ShapeDtypeStruct((NP, D2), jnp.float32),
    )(part2, r2, invdeg, bl2r)
    return out[:N]


# speedup vs baseline: 13.4188x; 1.1568x over previous
import functools

import jax
import jax.numpy as jnp
from jax import lax
from jax.experimental import pallas as pl
from jax.experimental.pallas import tpu as pltpu
from jax.experimental.pallas import tpu_sc as plsc

N = 10000
E = 320000
IN_DIM = 128
HID = 64
OUT = 16
EPS = 1e-5

NP = 10240
FILL = N
D1 = 80
D2 = OUT
B = 128
NC = 2
NS = 16
NWORK = NC * NS
G = 80
E_PAD = NWORK * G * B
ROWS_PER_TILE = NP // NS
ZB = ROWS_PER_TILE // B


def _make_sc_aggregate(d):
    mesh = plsc.VectorSubcoreMesh(core_axis_name="c", subcore_axis_name="s")

    def body(p_hbm, src_hbm, dst_hbm, z_hbm, out_hbm,
             src_v, dst_v, rows0_v, rows1_v, zbuf_v, acc_sh, sem0, sem1):
        cid = lax.axis_index("c")
        sid = lax.axis_index("s")
        tile = cid * NS + sid

        pltpu.sync_copy(z_hbm, zbuf_v)
        for j in range(ZB):
            pltpu.sync_copy(zbuf_v, acc_sh.at[pl.ds(sid * ROWS_PER_TILE + j * B, B)])
        plsc.subcore_barrier()

        base = tile * G
        pltpu.sync_copy(src_hbm.at[pl.ds(base, G)], src_v)
        pltpu.sync_copy(dst_hbm.at[pl.ds(base, G)], dst_v)

        pltpu.async_copy(p_hbm.at[src_v.at[0]], rows0_v, sem0)

        def step(i, carry):
            g = 2 * i
            pltpu.make_async_copy(p_hbm.at[src_v.at[g]], rows0_v, sem0).wait()
            pltpu.async_copy(p_hbm.at[src_v.at[g + 1]], rows1_v, sem1)
            pltpu.sync_copy(rows0_v, acc_sh.at[dst_v.at[g]], add=True)
            pltpu.make_async_copy(p_hbm.at[src_v.at[g + 1]], rows1_v, sem1).wait()

            @pl.when(i + 1 < G // 2)
            def _():
                pltpu.async_copy(p_hbm.at[src_v.at[g + 2]], rows0_v, sem0)

            pltpu.sync_copy(rows1_v, acc_sh.at[dst_v.at[g + 1]], add=True)
            return carry

        lax.fori_loop(0, G // 2, step, 0)
        plsc.subcore_barrier()

        for j in range(ZB):
            r0 = sid * ROWS_PER_TILE + j * B
            pltpu.sync_copy(acc_sh.at[pl.ds(r0, B)], rows0_v)
            pltpu.sync_copy(rows0_v, out_hbm.at[cid, pl.ds(r0, B)])

    return functools.partial(
        pl.kernel,
        out_type=jax.ShapeDtypeStruct((NC, NP, d), jnp.float32),
        mesh=mesh,
        scratch_types=[
            pltpu.VMEM((G, B), jnp.int32),
            pltpu.VMEM((G, B), jnp.int32),
            pltpu.VMEM((B, d), jnp.float32),
            pltpu.VMEM((B, d), jnp.float32),
            pltpu.VMEM((B, d), jnp.float32),
            pltpu.VMEM_SHARED((NP, d), jnp.float32),
            pltpu.SemaphoreType.DMA,
            pltpu.SemaphoreType.DMA,
        ],
        compiler_params=pltpu.CompilerParams(use_tc_tiling_on_sc=False),
    )(body)


_sc_agg1 = _make_sc_aggregate(D1)
_sc_agg2 = _make_sc_aggregate(D2)

_DN = (((1,), (1,)), ((), ()))


def _tc_pre(x_ref, wl_ref, wr_ref, p1_ref, r1_ref):
    x = x_ref[...]
    xw = lax.dot_general(x, wl_ref[...], _DN, preferred_element_type=jnp.float32)
    r1_ref[...] = lax.dot_general(x, wr_ref[...], _DN,
                                  preferred_element_type=jnp.float32)
    ones = jnp.ones((NP, 1), jnp.float32)
    zeros = jnp.zeros((NP, D1 - HID - 1), jnp.float32)
    p1_ref[...] = jnp.concatenate([xw, ones, zeros], axis=1)


def _tc_mid(part_ref, r1_ref, bl1_ref, gamma_ref, beta_ref, wl2_ref, wr2_ref,
            p2_ref, r2_ref, invdeg_ref):
    acc = part_ref[0] + part_ref[1]
    deg = acc[:, HID:HID + 1]
    invdeg = 1.0 / jnp.maximum(deg, 1.0)
    invdeg_ref[...] = invdeg
    h = acc[:, :HID] * invdeg + bl1_ref[...] + r1_ref[...]
    rows = lax.broadcasted_iota(jnp.int32, (NP, 1), 0)
    mask = rows < N
    hm = jnp.where(mask, h, 0.0)
    mean = jnp.sum(hm, axis=0, keepdims=True) * (1.0 / N)
    cm = jnp.where(mask, h - mean, 0.0)
    var = jnp.sum(cm * cm, axis=0, keepdims=True) * (1.0 / N)
    hn = (h - mean) * lax.rsqrt(var + EPS) * gamma_ref[...] + beta_ref[...]
    hr = jnp.maximum(hn, 0.0)
    p2_ref[...] = lax.dot_general(hr, wl2_ref[...], _DN,
                                  preferred_element_type=jnp.float32)
    r2_ref[...] = lax.dot_general(hr, wr2_ref[...], _DN,
                                  preferred_element_type=jnp.float32)


def _tc_post(part2_ref, r2_ref, invdeg_ref, bl2_ref, out_ref):
    acc = part2_ref[0] + part2_ref[1]
    out_ref[...] = acc * invdeg_ref[...] + bl2_ref[...] + r2_ref[...]


def kernel(x, edge_index, Wl1, bl1, Wr1, gamma, beta, Wl2, bl2, Wr2):
    xp = jnp.pad(x, ((0, NP - N), (0, 0)))
    src = edge_index[0].astype(jnp.int32)
    dst = edge_index[1].astype(jnp.int32)
    fill = N + (jnp.arange(E_PAD - E, dtype=jnp.int32) % (NP - N))
    src2d = jnp.concatenate([src, fill]).reshape(NWORK * G, B)
    dst2d = jnp.concatenate([dst, fill]).reshape(NWORK * G, B)
    z1 = jnp.zeros((B, D1), jnp.float32)
    z2 = jnp.zeros((B, D2), jnp.float32)
    bl1r = bl1.reshape(1, HID)
    gammar = gamma.reshape(1, HID)
    betar = beta.reshape(1, HID)
    bl2r = bl2.reshape(1, OUT)

    p1ext, r1 = pl.pallas_call(
        _tc_pre,
        out_shape=[jax.ShapeDtypeStruct((NP, D1), jnp.float32),
                   jax.ShapeDtypeStruct((NP, HID), jnp.float32)],
    )(xp, Wl1, Wr1)

    part1 = _sc_agg1(p1ext, src2d, dst2d, z1)

    p2, r2, invdeg = pl.pallas_call(
        _tc_mid,
        out_shape=[jax.ShapeDtypeStruct((NP, D2), jnp.float32),
                   jax.ShapeDtypeStruct((NP, D2), jnp.float32),
                   jax.ShapeDtypeStruct((NP, 1), jnp.float32)],
    )(part1, r1, bl1r, gammar, betar, Wl2, Wr2)

    part2 = _sc_agg2(p2, src2d, dst2d, z2)

    out = pl.pallas_call(
        _tc_post,
        out_shape=jax.ShapeDtypeStruct((NP, D2), jnp.float32),
    )(part2, r2, invdeg, bl2r)
    return out[:N]


# speedup vs baseline: 18.7940x; 1.4006x over previous
import functools

import jax
import jax.numpy as jnp
from jax import lax
from jax.experimental import pallas as pl
from jax.experimental.pallas import tpu as pltpu
from jax.experimental.pallas import tpu_sc as plsc

N = 10000
E = 320000
IN_DIM = 128
HID = 64
OUT = 16
EPS = 1e-5

NP = 10240
FILL = N
D1 = 80
D2 = OUT
B = 128
NC = 2
NS = 16
NWORK = NC * NS
G = 80
E_PAD = NWORK * G * B
ROWS_PER_TILE = NP // NS
ZB = ROWS_PER_TILE // B
NBUF = 4


def _make_sc_aggregate(d):
    mesh = plsc.VectorSubcoreMesh(core_axis_name="c", subcore_axis_name="s")

    def body(p_hbm, src_hbm, dst_hbm, z_hbm, out_hbm,
             src_v, dst_v, r0, r1, r2, r3, zbuf_v, acc_sh,
             g0, g1, g2, g3, s0, s1, s2, s3):
        rows = [r0, r1, r2, r3]
        gsem = [g0, g1, g2, g3]
        ssem = [s0, s1, s2, s3]
        cid = lax.axis_index("c")
        sid = lax.axis_index("s")
        tile = cid * NS + sid

        pltpu.sync_copy(z_hbm, zbuf_v)
        for j in range(ZB):
            pltpu.sync_copy(zbuf_v, acc_sh.at[pl.ds(sid * ROWS_PER_TILE + j * B, B)])
        plsc.subcore_barrier()

        base = tile * G
        pltpu.sync_copy(src_hbm.at[pl.ds(base, G)], src_v)
        pltpu.sync_copy(dst_hbm.at[pl.ds(base, G)], dst_v)

        for k in range(NBUF):
            pltpu.async_copy(p_hbm.at[src_v.at[k]], rows[k], gsem[k])

        n_iter = G // NBUF

        def step(i, carry):
            for k in range(NBUF):
                g = NBUF * i + k
                pltpu.make_async_copy(p_hbm.at[src_v.at[g]], rows[k],
                                      gsem[k]).wait()
                pltpu.async_copy(rows[k], acc_sh.at[dst_v.at[g]], ssem[k],
                                 add=True)

                @pl.when(i < n_iter - 1)
                def _():
                    pltpu.make_async_copy(rows[k], acc_sh.at[dst_v.at[g]],
                                          ssem[k]).wait()
                    pltpu.async_copy(p_hbm.at[src_v.at[g + NBUF]], rows[k],
                                     gsem[k])

            return carry

        lax.fori_loop(0, n_iter, step, 0)
        for k in range(NBUF):
            pltpu.make_async_copy(rows[k], acc_sh.at[dst_v.at[G - NBUF + k]],
                                  ssem[k]).wait()
        plsc.subcore_barrier()

        for j in range(ZB):
            base_r = sid * ROWS_PER_TILE + j * B
            pltpu.sync_copy(acc_sh.at[pl.ds(base_r, B)], rows[j % NBUF])
            pltpu.sync_copy(rows[j % NBUF], out_hbm.at[cid, pl.ds(base_r, B)])

    return functools.partial(
        pl.kernel,
        out_type=jax.ShapeDtypeStruct((NC, NP, d), jnp.float32),
        mesh=mesh,
        scratch_types=[
            pltpu.VMEM((G, B), jnp.int32),
            pltpu.VMEM((G, B), jnp.int32),
            pltpu.VMEM((B, d), jnp.float32),
            pltpu.VMEM((B, d), jnp.float32),
            pltpu.VMEM((B, d), jnp.float32),
            pltpu.VMEM((B, d), jnp.float32),
            pltpu.VMEM((B, d), jnp.float32),
            pltpu.VMEM_SHARED((NP, d), jnp.float32),
            pltpu.SemaphoreType.DMA,
            pltpu.SemaphoreType.DMA,
            pltpu.SemaphoreType.DMA,
            pltpu.SemaphoreType.DMA,
            pltpu.SemaphoreType.DMA,
            pltpu.SemaphoreType.DMA,
            pltpu.SemaphoreType.DMA,
            pltpu.SemaphoreType.DMA,
        ],
        compiler_params=pltpu.CompilerParams(use_tc_tiling_on_sc=False),
    )(body)


_sc_agg1 = _make_sc_aggregate(D1)
_sc_agg2 = _make_sc_aggregate(D2)

_DN = (((1,), (1,)), ((), ()))


def _tc_pre(x_ref, wl_ref, wr_ref, p1_ref, r1_ref):
    x = x_ref[...]
    xw = lax.dot_general(x, wl_ref[...], _DN, preferred_element_type=jnp.float32)
    r1_ref[...] = lax.dot_general(x, wr_ref[...], _DN,
                                  preferred_element_type=jnp.float32)
    ones = jnp.ones((NP, 1), jnp.float32)
    zeros = jnp.zeros((NP, D1 - HID - 1), jnp.float32)
    p1_ref[...] = jnp.concatenate([xw, ones, zeros], axis=1)


def _tc_mid(part_ref, r1_ref, bl1_ref, gamma_ref, beta_ref, wl2_ref, wr2_ref,
            p2_ref, r2_ref, invdeg_ref):
    acc = part_ref[0] + part_ref[1]
    deg = acc[:, HID:HID + 1]
    invdeg = 1.0 / jnp.maximum(deg, 1.0)
    invdeg_ref[...] = invdeg
    h = acc[:, :HID] * invdeg + bl1_ref[...] + r1_ref[...]
    rows = lax.broadcasted_iota(jnp.int32, (NP, 1), 0)
    mask = rows < N
    hm = jnp.where(mask, h, 0.0)
    mean = jnp.sum(hm, axis=0, keepdims=True) * (1.0 / N)
    cm = jnp.where(mask, h - mean, 0.0)
    var = jnp.sum(cm * cm, axis=0, keepdims=True) * (1.0 / N)
    hn = (h - mean) * lax.rsqrt(var + EPS) * gamma_ref[...] + beta_ref[...]
    hr = jnp.maximum(hn, 0.0)
    p2_ref[...] = lax.dot_general(hr, wl2_ref[...], _DN,
                                  preferred_element_type=jnp.float32)
    r2_ref[...] = lax.dot_general(hr, wr2_ref[...], _DN,
                                  preferred_element_type=jnp.float32)


def _tc_post(part2_ref, r2_ref, invdeg_ref, bl2_ref, out_ref):
    acc = part2_ref[0] + part2_ref[1]
    out_ref[...] = acc * invdeg_ref[...] + bl2_ref[...] + r2_ref[...]


def kernel(x, edge_index, Wl1, bl1, Wr1, gamma, beta, Wl2, bl2, Wr2):
    xp = jnp.pad(x, ((0, NP - N), (0, 0)))
    src = edge_index[0].astype(jnp.int32)
    dst = edge_index[1].astype(jnp.int32)
    fill = N + (jnp.arange(E_PAD - E, dtype=jnp.int32) % (NP - N))
    src2d = jnp.concatenate([src, fill]).reshape(NWORK * G, B)
    dst2d = jnp.concatenate([dst, fill]).reshape(NWORK * G, B)
    z1 = jnp.zeros((B, D1), jnp.float32)
    z2 = jnp.zeros((B, D2), jnp.float32)
    bl1r = bl1.reshape(1, HID)
    gammar = gamma.reshape(1, HID)
    betar = beta.reshape(1, HID)
    bl2r = bl2.reshape(1, OUT)

    p1ext, r1 = pl.pallas_call(
        _tc_pre,
        out_shape=[jax.ShapeDtypeStruct((NP, D1), jnp.float32),
                   jax.ShapeDtypeStruct((NP, HID), jnp.float32)],
    )(xp, Wl1, Wr1)

    part1 = _sc_agg1(p1ext, src2d, dst2d, z1)

    p2, r2, invdeg = pl.pallas_call(
        _tc_mid,
        out_shape=[jax.ShapeDtypeStruct((NP, D2), jnp.float32),
                   jax.ShapeDtypeStruct((NP, D2), jnp.float32),
                   jax.ShapeDtypeStruct((NP, 1), jnp.float32)],
    )(part1, r1, bl1r, gammar, betar, Wl2, Wr2)

    part2 = _sc_agg2(p2, src2d, dst2d, z2)

    out = pl.pallas_call(
        _tc_post,
        out_shape=jax.ShapeDtypeStruct((NP, D2), jnp.float32),
    )(part2, r2, invdeg, bl2r)
    return out[:N]


# speedup vs baseline: 20.6452x; 1.0985x over previous
import functools

import jax
import jax.numpy as jnp
from jax import lax
from jax.experimental import pallas as pl
from jax.experimental.pallas import tpu as pltpu
from jax.experimental.pallas import tpu_sc as plsc

N = 10000
E = 320000
IN_DIM = 128
HID = 64
OUT = 16
EPS = 1e-5

NP = 10240
FILL = N
D1 = 80
D2 = OUT
B = 128
NC = 2
NS = 16
NWORK = NC * NS
G = 80
E_PAD = NWORK * G * B
ROWS_PER_TILE = NP // NS
ZB = ROWS_PER_TILE // B
NBUF = 4


def _make_sc_aggregate(d):
    mesh = plsc.VectorSubcoreMesh(core_axis_name="c", subcore_axis_name="s")

    def body(p_hbm, src_hbm, dst_hbm, z_hbm, out_hbm,
             src_v, dst_v, r0, r1, r2, r3, zbuf_v, acc_sh,
             g0, g1, g2, g3, s0, s1, s2, s3):
        rows = [r0, r1, r2, r3]
        gsem = [g0, g1, g2, g3]
        ssem = [s0, s1, s2, s3]
        cid = lax.axis_index("c")
        sid = lax.axis_index("s")
        tile = cid * NS + sid

        pltpu.sync_copy(z_hbm, zbuf_v)
        for j in range(ZB):
            pltpu.sync_copy(zbuf_v, acc_sh.at[pl.ds(sid * ROWS_PER_TILE + j * B, B)])
        plsc.subcore_barrier()

        base = tile * G
        pltpu.sync_copy(src_hbm.at[pl.ds(base, G)], src_v)
        pltpu.sync_copy(dst_hbm.at[pl.ds(base, G)], dst_v)

        for k in range(NBUF):
            pltpu.async_copy(p_hbm.at[src_v.at[k]], rows[k], gsem[k])

        n_iter = G // NBUF

        def step(i, carry):
            for k in range(NBUF):
                g = NBUF * i + k
                pltpu.make_async_copy(p_hbm.at[src_v.at[g]], rows[k],
                                      gsem[k]).wait()
                pltpu.async_copy(rows[k], acc_sh.at[dst_v.at[g]], ssem[k],
                                 add=True)

                @pl.when(i < n_iter - 1)
                def _():
                    pltpu.make_async_copy(rows[k], acc_sh.at[dst_v.at[g]],
                                          ssem[k]).wait()
                    pltpu.async_copy(p_hbm.at[src_v.at[g + NBUF]], rows[k],
                                     gsem[k])

            return carry

        lax.fori_loop(0, n_iter, step, 0)
        for k in range(NBUF):
            pltpu.make_async_copy(rows[k], acc_sh.at[dst_v.at[G - NBUF + k]],
                                  ssem[k]).wait()
        plsc.subcore_barrier()

        for j in range(ZB):
            base_r = sid * ROWS_PER_TILE + j * B
            pltpu.sync_copy(acc_sh.at[pl.ds(base_r, B)], rows[j % NBUF])
            pltpu.sync_copy(rows[j % NBUF], out_hbm.at[cid, pl.ds(base_r, B)])

    return functools.partial(
        pl.kernel,
        out_type=jax.ShapeDtypeStruct((NC, NP, d), jnp.float32),
        mesh=mesh,
        scratch_types=[
            pltpu.VMEM((G, B), jnp.int32),
            pltpu.VMEM((G, B), jnp.int32),
            pltpu.VMEM((B, d), jnp.float32),
            pltpu.VMEM((B, d), jnp.float32),
            pltpu.VMEM((B, d), jnp.float32),
            pltpu.VMEM((B, d), jnp.float32),
            pltpu.VMEM((B, d), jnp.float32),
            pltpu.VMEM_SHARED((NP, d), jnp.float32),
            pltpu.SemaphoreType.DMA,
            pltpu.SemaphoreType.DMA,
            pltpu.SemaphoreType.DMA,
            pltpu.SemaphoreType.DMA,
            pltpu.SemaphoreType.DMA,
            pltpu.SemaphoreType.DMA,
            pltpu.SemaphoreType.DMA,
            pltpu.SemaphoreType.DMA,
        ],
        compiler_params=pltpu.CompilerParams(use_tc_tiling_on_sc=False),
    )(body)


_sc_agg1 = _make_sc_aggregate(D1)
_sc_agg2 = _make_sc_aggregate(D2)

_DN = (((1,), (1,)), ((), ()))


NG = E // B
NGP = E_PAD // B


def _tc_pre(x_ref, wl_ref, wr_ref, ei_ref, p1_ref, r1_ref, src_ref, dst_ref):
    x = x_ref[...]
    xw = lax.dot_general(x, wl_ref[...], _DN, preferred_element_type=jnp.float32)
    xr = lax.dot_general(x, wr_ref[...], _DN, preferred_element_type=jnp.float32)
    r1_ref[...] = jnp.concatenate(
        [xr, jnp.zeros((NP - N, HID), jnp.float32)], axis=0)
    ones = jnp.ones((N, 1), jnp.float32)
    zeros = jnp.zeros((N, D1 - HID - 1), jnp.float32)
    top = jnp.concatenate([xw, ones, zeros], axis=1)
    pad_row = jnp.concatenate(
        [jnp.zeros((NP - N, HID), jnp.float32),
         jnp.ones((NP - N, 1), jnp.float32),
         jnp.zeros((NP - N, D1 - HID - 1), jnp.float32)], axis=1)
    p1_ref[...] = jnp.concatenate([top, pad_row], axis=0)
    i2 = (lax.broadcasted_iota(jnp.int32, (NGP - NG, B), 0) * B
          + lax.broadcasted_iota(jnp.int32, (NGP - NG, B), 1))
    fill = N + lax.rem(i2, jnp.int32(NP - N))
    src_ref[...] = jnp.concatenate(
        [ei_ref[0].reshape(NG, B), fill], axis=0)
    dst_ref[...] = jnp.concatenate(
        [ei_ref[1].reshape(NG, B), fill], axis=0)


def _tc_mid(part_ref, r1_ref, bl1_ref, gamma_ref, beta_ref, wl2_ref, wr2_ref,
            p2_ref, r2_ref, invdeg_ref):
    acc = part_ref[0] + part_ref[1]
    deg = acc[:, HID:HID + 1]
    invdeg = 1.0 / jnp.maximum(deg, 1.0)
    invdeg_ref[...] = invdeg
    h = acc[:, :HID] * invdeg + bl1_ref[...] + r1_ref[...]
    rows = lax.broadcasted_iota(jnp.int32, (NP, 1), 0)
    mask = rows < N
    hm = jnp.where(mask, h, 0.0)
    mean = jnp.sum(hm, axis=0, keepdims=True) * (1.0 / N)
    cm = jnp.where(mask, h - mean, 0.0)
    var = jnp.sum(cm * cm, axis=0, keepdims=True) * (1.0 / N)
    hn = (h - mean) * lax.rsqrt(var + EPS) * gamma_ref[...] + beta_ref[...]
    hr = jnp.maximum(hn, 0.0)
    p2_ref[...] = lax.dot_general(hr, wl2_ref[...], _DN,
                                  preferred_element_type=jnp.float32)
    r2_ref[...] = lax.dot_general(hr, wr2_ref[...], _DN,
                                  preferred_element_type=jnp.float32)


def _tc_post(part2_ref, r2_ref, invdeg_ref, bl2_ref, out_ref):
    acc = part2_ref[0] + part2_ref[1]
    val = acc * invdeg_ref[...] + bl2_ref[...] + r2_ref[...]
    out_ref[...] = val[:N]


def kernel(x, edge_index, Wl1, bl1, Wr1, gamma, beta, Wl2, bl2, Wr2):
    ei = edge_index.astype(jnp.int32)
    z1 = jnp.zeros((B, D1), jnp.float32)
    z2 = jnp.zeros((B, D2), jnp.float32)
    bl1r = bl1.reshape(1, HID)
    gammar = gamma.reshape(1, HID)
    betar = beta.reshape(1, HID)
    bl2r = bl2.reshape(1, OUT)

    p1ext, r1, src2d, dst2d = pl.pallas_call(
        _tc_pre,
        out_shape=[jax.ShapeDtypeStruct((NP, D1), jnp.float32),
                   jax.ShapeDtypeStruct((NP, HID), jnp.float32),
                   jax.ShapeDtypeStruct((NGP, B), jnp.int32),
                   jax.ShapeDtypeStruct((NGP, B), jnp.int32)],
    )(x, Wl1, Wr1, ei)

    part1 = _sc_agg1(p1ext, src2d, dst2d, z1)

    p2, r2, invdeg = pl.pallas_call(
        _tc_mid,
        out_shape=[jax.ShapeDtypeStruct((NP, D2), jnp.float32),
                   jax.ShapeDtypeStruct((NP, D2), jnp.float32),
                   jax.ShapeDtypeStruct((NP, 1), jnp.float32)],
    )(part1, r1, bl1r, gammar, betar, Wl2, Wr2)

    part2 = _sc_agg2(p2, src2d, dst2d, z2)

    out = pl.pallas_call(
        _tc_post,
        out_shape=jax.ShapeDtypeStruct((N, D2), jnp.float32),
    )(part2, r2, invdeg, bl2r)
    return out


# speedup vs baseline: 20.7206x; 1.0037x over previous
import functools

import jax
import jax.numpy as jnp
from jax import lax
from jax.experimental import pallas as pl
from jax.experimental.pallas import tpu as pltpu
from jax.experimental.pallas import tpu_sc as plsc

N = 10000
E = 320000
IN_DIM = 128
HID = 64
OUT = 16
EPS = 1e-5

NP = 10240
FILL = N
D1 = HID
D2 = OUT
B = 128
NC = 2
NS = 16
NWORK = NC * NS
G = 80
E_PAD = NWORK * G * B
ROWS_PER_TILE = NP // NS
ZB = ROWS_PER_TILE // B
NBUF = 4


DD = 16


def _make_sc_aggregate(d, nbuf, with_deg):
    mesh = plsc.VectorSubcoreMesh(core_axis_name="c", subcore_axis_name="s")
    n_iter = G // nbuf

    def body(*refs):
        it = iter(refs)
        p_hbm = next(it); src_hbm = next(it); dst_hbm = next(it)
        z_hbm = next(it)
        if with_deg:
            zd_hbm = next(it); ones_hbm = next(it)
        out_hbm = next(it)
        if with_deg:
            deg_hbm = next(it)
        src_v = next(it); dst_v = next(it)
        rows = [next(it) for _ in range(nbuf)]
        zbuf_v = next(it)
        if with_deg:
            ones_v = next(it); dbuf_v = next(it)
        acc_sh = next(it)
        if with_deg:
            deg_sh = next(it)
        gsem = [next(it) for _ in range(nbuf)]
        ssem = [next(it) for _ in range(nbuf)]
        if with_deg:
            dsem = [next(it) for _ in range(nbuf)]

        cid = lax.axis_index("c")
        sid = lax.axis_index("s")
        tile = cid * NS + sid

        pltpu.sync_copy(z_hbm, zbuf_v)
        if with_deg:
            pltpu.sync_copy(zd_hbm, dbuf_v)
            pltpu.sync_copy(ones_hbm, ones_v)
        for j in range(ZB):
            r0 = sid * ROWS_PER_TILE + j * B
            pltpu.sync_copy(zbuf_v, acc_sh.at[pl.ds(r0, B)])
            if with_deg:
                pltpu.sync_copy(dbuf_v, deg_sh.at[pl.ds(r0, B)])
        plsc.subcore_barrier()

        base = tile * G
        pltpu.sync_copy(src_hbm.at[pl.ds(base, G)], src_v)
        pltpu.sync_copy(dst_hbm.at[pl.ds(base, G)], dst_v)

        for k in range(nbuf):
            pltpu.async_copy(p_hbm.at[src_v.at[k]], rows[k], gsem[k])

        def step(i, carry):
            for k in range(nbuf):
                g = nbuf * i + k
                pltpu.make_async_copy(p_hbm.at[src_v.at[g]], rows[k],
                                      gsem[k]).wait()
                pltpu.async_copy(rows[k], acc_sh.at[dst_v.at[g]], ssem[k],
                                 add=True)
                if with_deg:
                    pltpu.async_copy(ones_v, deg_sh.at[dst_v.at[g]], dsem[k],
                                     add=True)

                @pl.when(i < n_iter - 1)
                def _():
                    pltpu.make_async_copy(rows[k], acc_sh.at[dst_v.at[g]],
                                          ssem[k]).wait()
                    if with_deg:
                        pltpu.make_async_copy(ones_v, deg_sh.at[dst_v.at[g]],
                                              dsem[k]).wait()
                    pltpu.async_copy(p_hbm.at[src_v.at[g + nbuf]], rows[k],
                                     gsem[k])

            return carry

        lax.fori_loop(0, n_iter, step, 0)
        for k in range(nbuf):
            g = G - nbuf + k
            pltpu.make_async_copy(rows[k], acc_sh.at[dst_v.at[g]],
                                  ssem[k]).wait()
            if with_deg:
                pltpu.make_async_copy(ones_v, deg_sh.at[dst_v.at[g]],
                                      dsem[k]).wait()
        plsc.subcore_barrier()

        for j in range(ZB):
            r0 = sid * ROWS_PER_TILE + j * B
            pltpu.sync_copy(acc_sh.at[pl.ds(r0, B)], rows[j % nbuf])
            pltpu.sync_copy(rows[j % nbuf], out_hbm.at[cid, pl.ds(r0, B)])
            if with_deg:
                pltpu.sync_copy(deg_sh.at[pl.ds(r0, B)], dbuf_v)
                pltpu.sync_copy(dbuf_v, deg_hbm.at[cid, pl.ds(r0, B)])

    out_type = [jax.ShapeDtypeStruct((NC, NP, d), jnp.float32)]
    if with_deg:
        out_type.append(jax.ShapeDtypeStruct((NC, NP, DD), jnp.float32))
    scratch = [
        pltpu.VMEM((G, B), jnp.int32),
        pltpu.VMEM((G, B), jnp.int32),
    ]
    scratch += [pltpu.VMEM((B, d), jnp.float32) for _ in range(nbuf)]
    scratch.append(pltpu.VMEM((B, d), jnp.float32))
    if with_deg:
        scratch.append(pltpu.VMEM((B, DD), jnp.float32))
        scratch.append(pltpu.VMEM((B, DD), jnp.float32))
    scratch.append(pltpu.VMEM_SHARED((NP, d), jnp.float32))
    if with_deg:
        scratch.append(pltpu.VMEM_SHARED((NP, DD), jnp.float32))
    n_sems = nbuf * (3 if with_deg else 2)
    scratch += [pltpu.SemaphoreType.DMA for _ in range(n_sems)]

    return functools.partial(
        pl.kernel,
        out_type=out_type,
        mesh=mesh,
        scratch_types=scratch,
        compiler_params=pltpu.CompilerParams(use_tc_tiling_on_sc=False),
    )(body)


_sc_agg1 = _make_sc_aggregate(D1, 5, True)
_sc_agg2 = _make_sc_aggregate(D2, 8, False)

_DN = (((1,), (1,)), ((), ()))


NG = E // B
NGP = E_PAD // B


def _tc_pre(x_ref, wl_ref, wr_ref, ei_ref, p1_ref, r1_ref, src_ref, dst_ref):
    x = x_ref[...]
    xw = lax.dot_general(x, wl_ref[...], _DN, preferred_element_type=jnp.float32)
    xr = lax.dot_general(x, wr_ref[...], _DN, preferred_element_type=jnp.float32)
    pad0 = jnp.zeros((NP - N, HID), jnp.float32)
    r1_ref[...] = jnp.concatenate([xr, pad0], axis=0)
    p1_ref[...] = jnp.concatenate([xw, pad0], axis=0)
    i2 = (lax.broadcasted_iota(jnp.int32, (NGP - NG, B), 0) * B
          + lax.broadcasted_iota(jnp.int32, (NGP - NG, B), 1))
    fill = N + lax.rem(i2, jnp.int32(NP - N))
    src_ref[...] = jnp.concatenate(
        [ei_ref[0].reshape(NG, B), fill], axis=0)
    dst_ref[...] = jnp.concatenate(
        [ei_ref[1].reshape(NG, B), fill], axis=0)


def _tc_mid(part_ref, degp_ref, r1_ref, bl1_ref, gamma_ref, beta_ref,
            wl2_ref, wr2_ref, p2_ref, r2_ref, invdeg_ref):
    acc = part_ref[0] + part_ref[1]
    deg = degp_ref[0, :, 0:1] + degp_ref[1, :, 0:1]
    invdeg = 1.0 / jnp.maximum(deg, 1.0)
    invdeg_ref[...] = invdeg
    h = acc * invdeg + bl1_ref[...] + r1_ref[...]
    rows = lax.broadcasted_iota(jnp.int32, (NP, 1), 0)
    mask = rows < N
    hm = jnp.where(mask, h, 0.0)
    mean = jnp.sum(hm, axis=0, keepdims=True) * (1.0 / N)
    cm = jnp.where(mask, h - mean, 0.0)
    var = jnp.sum(cm * cm, axis=0, keepdims=True) * (1.0 / N)
    hn = (h - mean) * lax.rsqrt(var + EPS) * gamma_ref[...] + beta_ref[...]
    hr = jnp.maximum(hn, 0.0)
    p2_ref[...] = lax.dot_general(hr, wl2_ref[...], _DN,
                                  preferred_element_type=jnp.float32)
    r2_ref[...] = lax.dot_general(hr, wr2_ref[...], _DN,
                                  preferred_element_type=jnp.float32)


def _tc_post(part2_ref, r2_ref, invdeg_ref, bl2_ref, out_ref):
    acc = part2_ref[0] + part2_ref[1]
    val = acc * invdeg_ref[...] + bl2_ref[...] + r2_ref[...]
    out_ref[...] = val[:N]


def kernel(x, edge_index, Wl1, bl1, Wr1, gamma, beta, Wl2, bl2, Wr2):
    ei = edge_index.astype(jnp.int32)
    z1 = jnp.zeros((B, D1), jnp.float32)
    z2 = jnp.zeros((B, D2), jnp.float32)
    zd = jnp.zeros((B, DD), jnp.float32)
    onesb = jnp.zeros((B, DD), jnp.float32).at[:, 0].set(1.0)
    bl1r = bl1.reshape(1, HID)
    gammar = gamma.reshape(1, HID)
    betar = beta.reshape(1, HID)
    bl2r = bl2.reshape(1, OUT)

    p1ext, r1, src2d, dst2d = pl.pallas_call(
        _tc_pre,
        out_shape=[jax.ShapeDtypeStruct((NP, D1), jnp.float32),
                   jax.ShapeDtypeStruct((NP, HID), jnp.float32),
                   jax.ShapeDtypeStruct((NGP, B), jnp.int32),
                   jax.ShapeDtypeStruct((NGP, B), jnp.int32)],
    )(x, Wl1, Wr1, ei)

    part1, degp = _sc_agg1(p1ext, src2d, dst2d, z1, zd, onesb)

    p2, r2, invdeg = pl.pallas_call(
        _tc_mid,
        out_shape=[jax.ShapeDtypeStruct((NP, D2), jnp.float32),
                   jax.ShapeDtypeStruct((NP, D2), jnp.float32),
                   jax.ShapeDtypeStruct((NP, 1), jnp.float32)],
    )(part1, degp, r1, bl1r, gammar, betar, Wl2, Wr2)

    (part2,) = _sc_agg2(p2, src2d, dst2d, z2)

    out = pl.pallas_call(
        _tc_post,
        out_shape=jax.ShapeDtypeStruct((N, D2), jnp.float32),
    )(part2, r2, invdeg, bl2r)
    return out
